# baseline (speedup 1.0000x reference)
import jax
import jax.numpy as jnp
from jax import lax
from jax.experimental import pallas as pl
from jax.experimental.pallas import tpu as pltpu


def _body(lp_ref, lt_ref, x_ref, y_ref, out_ref, acc_ref):
    i = pl.program_id(0)

    @pl.when(i == 0)
    def _init():
        acc_ref[0] = 0.0
        acc_ref[1] = 0.0
        acc_ref[2] = 0.0

    y = y_ref[0]
    t = (y == 1).astype(jnp.float32)
    pos = (y > 0).astype(jnp.float32)
    posneg = (y > -1).astype(jnp.float32)

    x = x_ref[0]
    z = 2.0 * x * (2.0 * t - 1.0) + 1.0
    neg_logpt = jnp.log(1.0 + jnp.exp(-jnp.abs(z))) - jnp.minimum(z, 0.0)
    w = 0.75 - 0.5 * t
    cls_part = 0.5 * jnp.sum(w * neg_logpt * posneg)

    d = lp_ref[0] - lt_ref[0]
    ad = jnp.abs(d)
    sl1 = jnp.where(ad < 1.0, 0.5 * d * d, ad - 0.5)
    j8 = lax.broadcasted_iota(jnp.int32, (1024, 128), 0) // 8
    ii = lax.broadcasted_iota(jnp.int32, (1024, 128), 1)
    e = (j8 == ii).astype(jnp.float32)
    s = jnp.dot(sl1, e, preferred_element_type=jnp.float32)
    loc_part = jnp.sum(s * pos)
    np_part = jnp.sum(pos)

    acc_ref[0] += loc_part
    acc_ref[1] += cls_part
    acc_ref[2] += np_part

    @pl.when(i == pl.num_programs(0) - 1)
    def _fin():
        inv = 1.0 / acc_ref[2]
        out_ref[0] = (0.2 * acc_ref[0] + acc_ref[1]) * inv


def kernel(loc_preds, loc_targets, cls_preds, cls_targets):
    b, a, dd = loc_preds.shape
    n = b * a
    rows = n // 128
    br = 250
    grid = rows // br
    lp = loc_preds.reshape(grid, br, 128 * dd)
    lt = loc_targets.reshape(grid, br, 128 * dd)
    x = cls_preds.reshape(grid, br, 128)
    y = cls_targets.reshape(grid, br, 128)

    out = pl.pallas_call(
        _body,
        grid=(grid,),
        in_specs=[
            pl.BlockSpec((1, br, 128 * dd), lambda i: (i, 0, 0)),
            pl.BlockSpec((1, br, 128 * dd), lambda i: (i, 0, 0)),
            pl.BlockSpec((1, br, 128), lambda i: (i, 0, 0)),
            pl.BlockSpec((1, br, 128), lambda i: (i, 0, 0)),
        ],
        out_specs=pl.BlockSpec(memory_space=pltpu.SMEM),
        out_shape=jax.ShapeDtypeStruct((1,), jnp.float32),
        scratch_shapes=[pltpu.SMEM((3,), jnp.float32)],
    )(lp, lt, x, y)
    return out[0]


# speedup vs baseline: 32.9020x; 32.9020x over previous
import jax
import jax.numpy as jnp
from jax import lax
from jax.experimental import pallas as pl
from jax.experimental.pallas import tpu as pltpu


def _body(lp_ref, lt_ref, x_ref, y_ref, out_ref, pos_ref, acc_ref):
    i = pl.program_id(0)
    nb = pl.num_programs(0)
    a = y_ref.shape[1]

    @pl.when(i == 0)
    def _cls():
        y = y_ref[...]
        lane = lax.broadcasted_iota(jnp.int32, y.shape, 1)
        valid = lane < a
        t = (valid & (y == 1)).astype(jnp.float32)
        pos = (valid & (y > 0)).astype(jnp.float32)
        x = x_ref[...].reshape(y.shape)
        z = 2.0 * x * (2.0 * t - 1.0) + 1.0
        neg_logpt = jnp.log(1.0 + jnp.exp(-jnp.abs(z))) - jnp.minimum(z, 0.0)
        w = 0.75 - 0.5 * t
        cls_elem = jnp.where(valid & (y > -1), w * neg_logpt, 0.0)
        acc_ref[0] = 0.0
        acc_ref[1] = 0.5 * jnp.sum(cls_elem)
        acc_ref[2] = jnp.sum(pos)
        pos_ref[...] = pos.reshape(pos_ref.shape)

    d = lp_ref[...] - lt_ref[...]
    ad = jnp.abs(d)
    sl1 = jnp.where(ad < 1.0, 0.5 * d * d, ad - 0.5)
    rs = jnp.sum(sl1, axis=1)
    lane1 = lax.broadcasted_iota(jnp.int32, rs.shape, 1)
    rs = jnp.where(lane1 < a, rs, 0.0)
    acc_ref[0] += jnp.sum(rs * pos_ref[i][...])

    @pl.when(i == nb - 1)
    def _fin():
        inv = 1.0 / acc_ref[2]
        out_ref[0] = (0.2 * acc_ref[0] + acc_ref[1]) * inv


def kernel(loc_preds, loc_targets, cls_preds, cls_targets):
    b, a, dd = loc_preds.shape
    lp = jnp.transpose(loc_preds, (0, 2, 1))
    lt = jnp.transpose(loc_targets, (0, 2, 1))
    x = jnp.transpose(cls_preds, (0, 2, 1))
    y = cls_targets

    out = pl.pallas_call(
        _body,
        grid=(b,),
        in_specs=[
            pl.BlockSpec((1, dd, a), lambda i: (i, 0, 0)),
            pl.BlockSpec((1, dd, a), lambda i: (i, 0, 0)),
            pl.BlockSpec((b, 1, a), lambda i: (0, 0, 0)),
            pl.BlockSpec((b, a), lambda i: (0, 0)),
        ],
        out_specs=pl.BlockSpec(memory_space=pltpu.SMEM),
        out_shape=jax.ShapeDtypeStruct((1,), jnp.float32),
        scratch_shapes=[
            pltpu.VMEM((b, 1, a), jnp.float32),
            pltpu.SMEM((3,), jnp.float32),
        ],
    )(lp, lt, x, y)
    return out[0]


# speedup vs baseline: 37.6500x; 1.1443x over previous
import jax
import jax.numpy as jnp
from jax import lax
from jax.experimental import pallas as pl
from jax.experimental.pallas import tpu as pltpu


def _body(lp_ref, lt_ref, x_ref, y_ref, out_ref, pos_ref, acc_ref):
    i = pl.program_id(0)
    nb = pl.num_programs(0)
    a = y_ref.shape[1]

    @pl.when(i == 0)
    def _cls():
        y = y_ref[...]
        lane = lax.broadcasted_iota(jnp.int32, y.shape, 1)
        valid = lane < a
        t = (valid & (y == 1)).astype(jnp.float32)
        pos = (valid & (y > 0)).astype(jnp.float32)
        x = x_ref[...].reshape(y.shape)
        z = 2.0 * x * (2.0 * t - 1.0) + 1.0
        neg_logpt = jnp.log(1.0 + jnp.exp(-jnp.abs(z))) - jnp.minimum(z, 0.0)
        w = 0.75 - 0.5 * t
        cls_elem = jnp.where(valid & (y > -1), w * neg_logpt, 0.0)
        acc_ref[0] = 0.0
        acc_ref[1] = 0.5 * jnp.sum(cls_elem)
        acc_ref[2] = jnp.sum(pos)
        pos_ref[...] = pos.reshape(pos_ref.shape)

    d = lp_ref[...] - lt_ref[...]
    ad = jnp.abs(d)
    q = jnp.minimum(ad, 1.0)
    sl1 = q * (ad - 0.5 * q)
    rs = jnp.sum(sl1.reshape(-1, 8, sl1.shape[-1]), axis=1)
    lane1 = lax.broadcasted_iota(jnp.int32, rs.shape, 1)
    rs = jnp.where(lane1 < a, rs, 0.0)
    bb = rs.shape[0]
    acc_ref[0] += jnp.sum(rs * pos_ref[pl.ds(i * bb, bb), 0, :])

    @pl.when(i == nb - 1)
    def _fin():
        inv = 1.0 / acc_ref[2]
        out_ref[0] = (0.2 * acc_ref[0] + acc_ref[1]) * inv


def kernel(loc_preds, loc_targets, cls_preds, cls_targets):
    b, a, dd = loc_preds.shape
    lp = jnp.transpose(loc_preds, (0, 2, 1)).reshape(b * dd, a)
    lt = jnp.transpose(loc_targets, (0, 2, 1)).reshape(b * dd, a)
    x = jnp.transpose(cls_preds, (0, 2, 1))
    y = cls_targets

    bb = 2
    out = pl.pallas_call(
        _body,
        grid=(b // bb,),
        in_specs=[
            pl.BlockSpec((bb * dd, a), lambda i: (i, 0)),
            pl.BlockSpec((bb * dd, a), lambda i: (i, 0)),
            pl.BlockSpec((b, 1, a), lambda i: (0, 0, 0)),
            pl.BlockSpec((b, a), lambda i: (0, 0)),
        ],
        out_specs=pl.BlockSpec(memory_space=pltpu.SMEM),
        out_shape=jax.ShapeDtypeStruct((1,), jnp.float32),
        scratch_shapes=[
            pltpu.VMEM((b, 1, a), jnp.float32),
            pltpu.SMEM((3,), jnp.float32),
        ],
    )(lp, lt, x, y)
    return out[0]


# speedup vs baseline: 38.4193x; 1.0204x over previous
import jax
import jax.numpy as jnp
from jax import lax
from jax.experimental import pallas as pl
from jax.experimental.pallas import tpu as pltpu

NBUF = 4
CHUNK = 8


def _body(lp_hbm, lt_hbm, x_ref, y_ref, out_ref,
          lpb, ltb, pos_ref, acc_ref, sems):
    a = y_ref.shape[1]
    nchunks = lp_hbm.shape[0] // CHUNK

    y = y_ref[...]
    lane = lax.broadcasted_iota(jnp.int32, y.shape, 1)
    valid = lane < a
    t = (valid & (y == 1)).astype(jnp.float32)
    pos = (valid & (y > 0)).astype(jnp.float32)
    x = x_ref[...].reshape(y.shape)
    z = 2.0 * x * (2.0 * t - 1.0) + 1.0
    neg_logpt = jnp.log(1.0 + jnp.exp(-jnp.abs(z))) - jnp.minimum(z, 0.0)
    w = 0.75 - 0.5 * t
    cls_elem = jnp.where(valid & (y > -1), w * neg_logpt, 0.0)
    cls_sum = 0.5 * jnp.sum(cls_elem)
    np_sum = jnp.sum(pos)
    pos_ref[...] = pos.reshape(pos_ref.shape)

    def start(c, slot):
        pltpu.make_async_copy(
            lp_hbm.at[pl.ds(c * CHUNK, CHUNK), :], lpb.at[slot], sems.at[slot, 0]
        ).start()
        pltpu.make_async_copy(
            lt_hbm.at[pl.ds(c * CHUNK, CHUNK), :], ltb.at[slot], sems.at[slot, 1]
        ).start()

    def wait(c, slot):
        pltpu.make_async_copy(
            lp_hbm.at[pl.ds(c * CHUNK, CHUNK), :], lpb.at[slot], sems.at[slot, 0]
        ).wait()
        pltpu.make_async_copy(
            lt_hbm.at[pl.ds(c * CHUNK, CHUNK), :], ltb.at[slot], sems.at[slot, 1]
        ).wait()

    for c in range(min(NBUF, nchunks)):
        start(c, c)

    def chunk_body(c, acc):
        slot = lax.rem(c, NBUF)
        wait(c, slot)
        d = lpb[slot] - ltb[slot]
        nxt = c + NBUF

        @pl.when(nxt < nchunks)
        def _():
            start(nxt, lax.rem(nxt, NBUF))

        ad = jnp.abs(d)
        q = jnp.minimum(ad, 1.0)
        sl1 = q * (ad - 0.5 * q)
        rs = jnp.sum(sl1.reshape(-1, 8, a), axis=1)
        lane1 = lax.broadcasted_iota(jnp.int32, rs.shape, 1)
        rs = jnp.where(lane1 < a, rs, 0.0)
        bb = rs.shape[0]
        pr = pos_ref[pl.ds(c * bb, bb), 0, :]
        return acc + jnp.sum(rs * pr)

    loc_sum = lax.fori_loop(0, nchunks, chunk_body, 0.0, unroll=False)
    acc_ref[0] = loc_sum
    inv = 1.0 / np_sum
    out_ref[0] = (0.2 * loc_sum + cls_sum) * inv


def kernel(loc_preds, loc_targets, cls_preds, cls_targets):
    b, a, dd = loc_preds.shape
    lp = jnp.transpose(loc_preds, (0, 2, 1)).reshape(b * dd, a)
    lt = jnp.transpose(loc_targets, (0, 2, 1)).reshape(b * dd, a)
    x = jnp.transpose(cls_preds, (0, 2, 1))
    y = cls_targets

    out = pl.pallas_call(
        _body,
        in_specs=[
            pl.BlockSpec(memory_space=pl.ANY),
            pl.BlockSpec(memory_space=pl.ANY),
            pl.BlockSpec((b, 1, a), lambda: (0, 0, 0)),
            pl.BlockSpec((b, a), lambda: (0, 0)),
        ],
        out_specs=pl.BlockSpec(memory_space=pltpu.SMEM),
        out_shape=jax.ShapeDtypeStruct((1,), jnp.float32),
        scratch_shapes=[
            pltpu.VMEM((NBUF, CHUNK, a), jnp.float32),
            pltpu.VMEM((NBUF, CHUNK, a), jnp.float32),
            pltpu.VMEM((b, 1, a), jnp.float32),
            pltpu.SMEM((1,), jnp.float32),
            pltpu.SemaphoreType.DMA((NBUF, 2)),
        ],
    )(lp, lt, x, y)
    return out[0]
